# baseline (speedup 1.0000x reference)
import functools

import jax
import jax.numpy as jnp
from jax import lax
from jax.experimental import pallas as pl
from jax.experimental.pallas import tpu as pltpu
from jax.experimental.pallas import tpu_sc as plsc

VOCAB = 32000
D = 2048
NTOK = 16384
EPS = 1.1920928955078125e-07

_info = plsc.get_sparse_core_info()
NC, NS, L = _info.num_cores, _info.num_subcores, _info.num_lanes
NW = NC * NS
ROWS_PER_W = NTOK // NW
CHUNK = 16
NCHUNKS = ROWS_PER_W // CHUNK
NBUF = 3
NSLICES = D // L


def _vrsqrt(a):
    i = lax.bitcast_convert_type(a, jnp.int32)
    i = jnp.int32(0x5F3759DF) - lax.shift_right_logical(i, 1)
    y = lax.bitcast_convert_type(i, jnp.float32)
    half = jnp.float32(0.5) * a
    for _ in range(2):
        y = y * (jnp.float32(1.5) - half * y * y)
    return y


def _chunk_scales(rows):
    scales = []
    for r in range(CHUNK):
        def acc_body(jj, accs, r=r):
            base = jj * (8 * L)
            out = []
            for u in range(8):
                v = rows[r, pl.ds(base + u * L, L)]
                out.append(accs[u] + v * v)
            return tuple(out)

        accs = lax.fori_loop(
            0, NSLICES // 8, acc_body,
            tuple(jnp.zeros((L,), jnp.float32) for _ in range(8)),
        )
        acc = ((accs[0] + accs[1]) + (accs[2] + accs[3])) + (
            (accs[4] + accs[5]) + (accs[6] + accs[7])
        )
        a = acc + lax.rev(acc, (0,))
        s = [a[i] for i in range(8)]
        total = ((s[0] + s[1]) + (s[2] + s[3])) + ((s[4] + s[5]) + (s[6] + s[7]))
        ms = total * jnp.float32(1.0 / D) + jnp.float32(EPS)
        scales.append(_vrsqrt(jnp.full((L,), ms, jnp.float32)))
    return tuple(scales)


def _sc_body(table_hbm, idx_hbm, w_hbm, out_hbm, idx_v, w_v, *bufs_and_sems):
    rows = list(bufs_and_sems[:NBUF])
    gsem = list(bufs_and_sems[NBUF:2 * NBUF])
    wsem = list(bufs_and_sems[2 * NBUF:3 * NBUF])

    wid = lax.axis_index("s") * NC + lax.axis_index("c")
    w_base = wid * ROWS_PER_W

    pltpu.sync_copy(w_hbm, w_v)
    pltpu.sync_copy(idx_hbm.at[pl.ds(w_base, ROWS_PER_W)], idx_v)

    def gather(g, b):
        pltpu.make_async_copy(
            table_hbm.at[idx_v.at[pl.ds(g * CHUNK, CHUNK)]], rows[b], gsem[b]
        ).start()

    def out_slice(g):
        return out_hbm.at[pl.ds(w_base + g * CHUNK, CHUNK)]

    gather(0, 0)

    def group_body(gp, carry):
        for b in range(NBUF):
            g = gp * NBUF + b
            nb = (b + 1) % NBUF
            @pl.when(g + 1 < NCHUNKS)
            def _fire():
                @pl.when(g + 1 >= NBUF)
                def _drain():
                    pltpu.make_async_copy(rows[nb], out_slice(g), wsem[nb]).wait()
                gather(g + 1, nb)

            pltpu.make_async_copy(
                table_hbm.at[idx_v.at[pl.ds(g * CHUNK, CHUNK)]], rows[b], gsem[b]
            ).wait()

            scales = _chunk_scales(rows[b])

            def norm_body(j, sc):
                for u in range(2):
                    sl = pl.ds((j * 2 + u) * L, L)
                    wv = w_v[sl]
                    for r in range(CHUNK):
                        rows[b][r, sl] = rows[b][r, sl] * sc[r] * wv
                return sc

            lax.fori_loop(0, NSLICES // 2, norm_body, scales)
            pltpu.make_async_copy(rows[b], out_slice(g), wsem[b]).start()
        return carry

    lax.fori_loop(0, NCHUNKS // NBUF, group_body, 0)
    for g in range((NCHUNKS // NBUF) * NBUF, NCHUNKS):
        b = g % NBUF
        nb = (b + 1) % NBUF
        if g + 1 < NCHUNKS:
            pltpu.make_async_copy(rows[nb], out_slice(g), wsem[nb]).wait()
            gather(g + 1, nb)
        pltpu.make_async_copy(
            table_hbm.at[idx_v.at[pl.ds(g * CHUNK, CHUNK)]], rows[b], gsem[b]
        ).wait()
        scales = _chunk_scales(rows[b])

        def tail_norm(j, sc):
            for u in range(2):
                sl = pl.ds((j * 2 + u) * L, L)
                wv = w_v[sl]
                for r in range(CHUNK):
                    rows[b][r, sl] = rows[b][r, sl] * sc[r] * wv
            return sc

        lax.fori_loop(0, NSLICES // 2, tail_norm, scales)
        pltpu.make_async_copy(rows[b], out_slice(g), wsem[b]).start()

    for g in range(max(0, NCHUNKS - NBUF), NCHUNKS):
        pltpu.make_async_copy(rows[g % NBUF], out_slice(g), wsem[g % NBUF]).wait()


@jax.jit
def _run(table, idx, w):
    mesh = plsc.VectorSubcoreMesh(core_axis_name="c", subcore_axis_name="s")
    scratch = [
        pltpu.VMEM((ROWS_PER_W,), jnp.int32),
        pltpu.VMEM((D,), jnp.float32),
    ]
    scratch += [pltpu.VMEM((CHUNK, D), jnp.float32) for _ in range(NBUF)]
    scratch += [pltpu.SemaphoreType.DMA for _ in range(2 * NBUF)]
    f = pl.kernel(
        _sc_body,
        mesh=mesh,
        out_type=jax.ShapeDtypeStruct((NTOK, D), jnp.float32),
        scratch_types=scratch,
    )
    return f(table, idx, w)


def kernel(x, table, rms_weight):
    b, s = x.shape
    idx = x.reshape(-1).astype(jnp.int32)
    out = _run(table, idx, rms_weight)
    return out.reshape(b, s, D)


# speedup vs baseline: 1.9572x; 1.9572x over previous
import functools

import jax
import jax.numpy as jnp
from jax import lax
from jax.experimental import pallas as pl
from jax.experimental.pallas import tpu as pltpu
from jax.experimental.pallas import tpu_sc as plsc

VOCAB = 32000
D = 2048
NTOK = 16384
EPS = 1.1920928955078125e-07

_info = plsc.get_sparse_core_info()
NC, NS, L = _info.num_cores, _info.num_subcores, _info.num_lanes
NW = NC * NS
ROWS_PER_W = NTOK // NW
CHUNK = 16
NCHUNKS = ROWS_PER_W // CHUNK
NBUF = 3
NSLICES = D // L


def _vrsqrt(a):
    i = lax.bitcast_convert_type(a, jnp.int32)
    i = jnp.int32(0x5F3759DF) - lax.shift_right_logical(i, 1)
    y = lax.bitcast_convert_type(i, jnp.float32)
    half = jnp.float32(0.5) * a
    for _ in range(2):
        y = y * (jnp.float32(1.5) - half * y * y)
    return y


def _chunk_scales(rows):
    scales = []
    for r in range(CHUNK):
        def acc_body(jj, accs, r=r):
            base = jj * (8 * L)
            out = []
            for u in range(8):
                v = rows[r, pl.ds(base + u * L, L)]
                out.append(accs[u] + v * v)
            return tuple(out)

        accs = lax.fori_loop(
            0, NSLICES // 8, acc_body,
            tuple(jnp.zeros((L,), jnp.float32) for _ in range(8)),
        )
        acc = ((accs[0] + accs[1]) + (accs[2] + accs[3])) + (
            (accs[4] + accs[5]) + (accs[6] + accs[7])
        )
        a = acc + lax.rev(acc, (0,))
        s = [a[i] for i in range(8)]
        total = ((s[0] + s[1]) + (s[2] + s[3])) + ((s[4] + s[5]) + (s[6] + s[7]))
        ms = total * jnp.float32(1.0 / D) + jnp.float32(EPS)
        scales.append(_vrsqrt(jnp.full((L,), ms, jnp.float32)))
    return tuple(scales)


def _sc_body(table_hbm, idx_hbm, w_hbm, out_hbm, idx_v, w_v, *bufs_and_sems):
    rows = list(bufs_and_sems[:NBUF])
    gsem = list(bufs_and_sems[NBUF:2 * NBUF])
    wsem = list(bufs_and_sems[2 * NBUF:3 * NBUF])

    wid = lax.axis_index("s") * NC + lax.axis_index("c")
    w_base = wid * ROWS_PER_W

    pltpu.sync_copy(w_hbm, w_v)
    pltpu.sync_copy(idx_hbm.at[pl.ds(w_base, ROWS_PER_W)], idx_v)

    def gather(g, b):
        pltpu.make_async_copy(
            table_hbm.at[idx_v.at[pl.ds(g * CHUNK, CHUNK)]], rows[b], gsem[b]
        ).start()

    def out_slice(g):
        return out_hbm.at[pl.ds(w_base + g * CHUNK, CHUNK)]

    gather(0, 0)

    def group_body(gp, carry):
        for b in range(NBUF):
            g = gp * NBUF + b
            nb = (b + 1) % NBUF
            @pl.when(g + 1 < NCHUNKS)
            def _fire():
                @pl.when(g + 1 >= NBUF)
                def _drain():
                    pltpu.make_async_copy(rows[nb], out_slice(g), wsem[nb]).wait()
                gather(g + 1, nb)

            pltpu.make_async_copy(
                table_hbm.at[idx_v.at[pl.ds(g * CHUNK, CHUNK)]], rows[b], gsem[b]
            ).wait()

            scales = _chunk_scales(rows[b])

            def norm_body(j, sc):
                sl = pl.ds(j * L, L)
                wv = w_v[sl]
                for r in range(CHUNK):
                    rows[b][r, sl] = rows[b][r, sl] * sc[r] * wv
                return sc

            lax.fori_loop(0, NSLICES, norm_body, scales)
            pltpu.make_async_copy(rows[b], out_slice(g), wsem[b]).start()
        return carry

    lax.fori_loop(0, NCHUNKS // NBUF, group_body, 0)
    for g in range((NCHUNKS // NBUF) * NBUF, NCHUNKS):
        b = g % NBUF
        nb = (b + 1) % NBUF
        if g + 1 < NCHUNKS:
            pltpu.make_async_copy(rows[nb], out_slice(g), wsem[nb]).wait()
            gather(g + 1, nb)
        pltpu.make_async_copy(
            table_hbm.at[idx_v.at[pl.ds(g * CHUNK, CHUNK)]], rows[b], gsem[b]
        ).wait()
        scales = _chunk_scales(rows[b])

        def tail_norm(j, sc):
            sl = pl.ds(j * L, L)
            wv = w_v[sl]
            for r in range(CHUNK):
                rows[b][r, sl] = rows[b][r, sl] * sc[r] * wv
            return sc

        lax.fori_loop(0, NSLICES, tail_norm, scales)
        pltpu.make_async_copy(rows[b], out_slice(g), wsem[b]).start()

    for g in range(max(0, NCHUNKS - NBUF), NCHUNKS):
        pltpu.make_async_copy(rows[g % NBUF], out_slice(g), wsem[g % NBUF]).wait()


@jax.jit
def _run(table, idx, w):
    mesh = plsc.VectorSubcoreMesh(core_axis_name="c", subcore_axis_name="s")
    scratch = [
        pltpu.VMEM((ROWS_PER_W,), jnp.int32),
        pltpu.VMEM((D,), jnp.float32),
    ]
    scratch += [pltpu.VMEM((CHUNK, D), jnp.float32) for _ in range(NBUF)]
    scratch += [pltpu.SemaphoreType.DMA for _ in range(2 * NBUF)]
    f = pl.kernel(
        _sc_body,
        mesh=mesh,
        out_type=jax.ShapeDtypeStruct((NTOK, D), jnp.float32),
        scratch_types=scratch,
    )
    return f(table, idx, w)


def kernel(x, table, rms_weight):
    b, s = x.shape
    idx = x.reshape(-1).astype(jnp.int32)
    out = _run(table, idx, rms_weight)
    return out.reshape(b, s, D)


# speedup vs baseline: 2.7924x; 1.4267x over previous
import functools

import jax
import jax.numpy as jnp
from jax import lax
from jax.experimental import pallas as pl
from jax.experimental.pallas import tpu as pltpu
from jax.experimental.pallas import tpu_sc as plsc

VOCAB = 32000
D = 2048
NTOK = 16384
EPS = 1.1920928955078125e-07

_info = plsc.get_sparse_core_info()
NC, NS, L = _info.num_cores, _info.num_subcores, _info.num_lanes
NW = NC * NS
ROWS_PER_W = NTOK // NW
CHUNK = 16
NCHUNKS = ROWS_PER_W // CHUNK
NBUF = 3
NSLICES = D // L


def _vrsqrt(a):
    i = lax.bitcast_convert_type(a, jnp.int32)
    i = jnp.int32(0x5F3759DF) - lax.shift_right_logical(i, 1)
    y = lax.bitcast_convert_type(i, jnp.float32)
    half = jnp.float32(0.5) * a
    for _ in range(2):
        y = y * (jnp.float32(1.5) - half * y * y)
    return y


def _chunk_scales(rows):
    scales = []
    for r in range(CHUNK):
        zeros = tuple(jnp.zeros((L,), jnp.float32) for _ in range(8))

        @plsc.parallel_loop(0, NSLICES // 8, carry=zeros)
        def accs(jj, accs, r=r):
            base = jj * (8 * L)
            out = []
            for u in range(8):
                v = rows[r, pl.ds(base + u * L, L)]
                out.append(accs[u] + v * v)
            return tuple(out)
        acc = ((accs[0] + accs[1]) + (accs[2] + accs[3])) + (
            (accs[4] + accs[5]) + (accs[6] + accs[7])
        )
        a = acc + lax.rev(acc, (0,))
        s = [a[i] for i in range(8)]
        total = ((s[0] + s[1]) + (s[2] + s[3])) + ((s[4] + s[5]) + (s[6] + s[7]))
        ms = total * jnp.float32(1.0 / D) + jnp.float32(EPS)
        scales.append(_vrsqrt(jnp.full((L,), ms, jnp.float32)))
    return tuple(scales)


def _sc_body(table_hbm, idx_hbm, w_hbm, out_hbm, idx_v, w_v, *bufs_and_sems):
    rows = list(bufs_and_sems[:NBUF])
    gsem = list(bufs_and_sems[NBUF:2 * NBUF])
    wsem = list(bufs_and_sems[2 * NBUF:3 * NBUF])

    wid = lax.axis_index("s") * NC + lax.axis_index("c")
    w_base = wid * ROWS_PER_W

    pltpu.sync_copy(w_hbm, w_v)
    pltpu.sync_copy(idx_hbm.at[pl.ds(w_base, ROWS_PER_W)], idx_v)

    def gather(g, b):
        pltpu.make_async_copy(
            table_hbm.at[idx_v.at[pl.ds(g * CHUNK, CHUNK)]], rows[b], gsem[b]
        ).start()

    def out_slice(g):
        return out_hbm.at[pl.ds(w_base + g * CHUNK, CHUNK)]

    gather(0, 0)

    def group_body(gp, carry):
        for b in range(NBUF):
            g = gp * NBUF + b
            nb = (b + 1) % NBUF
            @pl.when(g + 1 < NCHUNKS)
            def _fire():
                @pl.when(g + 1 >= NBUF)
                def _drain():
                    pltpu.make_async_copy(rows[nb], out_slice(g), wsem[nb]).wait()
                gather(g + 1, nb)

            pltpu.make_async_copy(
                table_hbm.at[idx_v.at[pl.ds(g * CHUNK, CHUNK)]], rows[b], gsem[b]
            ).wait()

            scales = _chunk_scales(rows[b])

            @plsc.parallel_loop(0, NSLICES, carry=scales)
            def _norm(j, sc):
                sl = pl.ds(j * L, L)
                wv = w_v[sl]
                for r in range(CHUNK):
                    rows[b][r, sl] = rows[b][r, sl] * sc[r] * wv
                return sc
            pltpu.make_async_copy(rows[b], out_slice(g), wsem[b]).start()
        return carry

    lax.fori_loop(0, NCHUNKS // NBUF, group_body, 0)
    for g in range((NCHUNKS // NBUF) * NBUF, NCHUNKS):
        b = g % NBUF
        nb = (b + 1) % NBUF
        if g + 1 < NCHUNKS:
            pltpu.make_async_copy(rows[nb], out_slice(g), wsem[nb]).wait()
            gather(g + 1, nb)
        pltpu.make_async_copy(
            table_hbm.at[idx_v.at[pl.ds(g * CHUNK, CHUNK)]], rows[b], gsem[b]
        ).wait()
        scales = _chunk_scales(rows[b])

        @plsc.parallel_loop(0, NSLICES, carry=scales)
        def _tail_norm(j, sc):
            sl = pl.ds(j * L, L)
            wv = w_v[sl]
            for r in range(CHUNK):
                rows[b][r, sl] = rows[b][r, sl] * sc[r] * wv
            return sc
        pltpu.make_async_copy(rows[b], out_slice(g), wsem[b]).start()

    for g in range(max(0, NCHUNKS - NBUF), NCHUNKS):
        pltpu.make_async_copy(rows[g % NBUF], out_slice(g), wsem[g % NBUF]).wait()


@jax.jit
def _run(table, idx, w):
    mesh = plsc.VectorSubcoreMesh(core_axis_name="c", subcore_axis_name="s")
    scratch = [
        pltpu.VMEM((ROWS_PER_W,), jnp.int32),
        pltpu.VMEM((D,), jnp.float32),
    ]
    scratch += [pltpu.VMEM((CHUNK, D), jnp.float32) for _ in range(NBUF)]
    scratch += [pltpu.SemaphoreType.DMA for _ in range(2 * NBUF)]
    f = pl.kernel(
        _sc_body,
        mesh=mesh,
        out_type=jax.ShapeDtypeStruct((NTOK, D), jnp.float32),
        scratch_types=scratch,
    )
    return f(table, idx, w)


def kernel(x, table, rms_weight):
    b, s = x.shape
    idx = x.reshape(-1).astype(jnp.int32)
    out = _run(table, idx, rms_weight)
    return out.reshape(b, s, D)
